# Initial kernel scaffold; baseline (speedup 1.0000x reference)
import jax
import jax.numpy as jnp
from jax.experimental import pallas as pl


def kernel(x, emb, pos, ln1_g, ln1_b, Wqk, Wv, Wout, bout, ln2_g, ln2_b, W1, b1, W2, b2, nf_g, nf_b, Wf, bf, Wc, bc):
    raise NotImplementedError("write your pallas kernel here")


# speedup vs baseline: 5.6161x; 5.6161x over previous
import jax
import jax.numpy as jnp
from jax.experimental import pallas as pl
from jax.experimental.pallas import tpu as pltpu
from jax.experimental.pallas import tpu_sc as plsc


def _sc_gather(emb, idx_flat):
    n = idx_flat.shape[0]
    dim = emb.shape[1]
    window = 128
    indices = idx_flat.reshape(1, n)
    mesh = plsc.VectorSubcoreMesh(core_axis_name="c", subcore_axis_name="s")

    @pl.kernel(out_type=jax.ShapeDtypeStruct((n, dim), emb.dtype), mesh=mesh)
    def gather_kernel(emb_hbm, i_hbm, o_hbm):
        def body(i_vmem, o_vmem):
            pltpu.sync_copy(emb_hbm.at[i_vmem.at[0]], o_vmem)

        pltpu.emit_pipeline(
            body,
            grid=(n // window,),
            in_specs=[pl.BlockSpec((1, window), lambda i: (0, i))],
            out_specs=[pl.BlockSpec((window, dim), lambda i: (i, 0))],
            core_axis_name=("c", "s"),
            dimension_semantics=(pltpu.PARALLEL,),
        )(i_hbm, o_hbm)

    return gather_kernel(emb, indices)


def _layernorm_in(x, g, b):
    mu = jnp.mean(x, axis=-1, keepdims=True)
    var = jnp.mean((x - mu) ** 2, axis=-1, keepdims=True)
    return (x - mu) * jax.lax.rsqrt(var + 1e-5) * g + b


def _embed(g, pos, g1, b1):
    bb, t, dim = g.shape

    def kern(g_ref, p_ref, g1_ref, b1_ref, h_ref, hn_ref):
        h = g_ref[0] + p_ref[...]
        h_ref[0] = h
        hn_ref[0] = _layernorm_in(h, g1_ref[...], b1_ref[...])

    return pl.pallas_call(
        kern,
        grid=(bb,),
        in_specs=[
            pl.BlockSpec((1, t, dim), lambda i: (i, 0, 0)),
            pl.BlockSpec((t, dim), lambda i: (0, 0)),
            pl.BlockSpec((1, dim), lambda i: (0, 0)),
            pl.BlockSpec((1, dim), lambda i: (0, 0)),
        ],
        out_specs=[
            pl.BlockSpec((1, t, dim), lambda i: (i, 0, 0)),
            pl.BlockSpec((1, t, dim), lambda i: (i, 0, 0)),
        ],
        out_shape=[
            jax.ShapeDtypeStruct((bb, t, dim), jnp.float32),
            jax.ShapeDtypeStruct((bb, t, dim), jnp.float32),
        ],
        compiler_params=pltpu.CompilerParams(
            dimension_semantics=("parallel",)),
    )(g, pos, g1, b1)


_JT = 1024


def _attention_block(hn, wqk_ref, wv_ref, wout_ref, nh, dh):
    t, dim = hn.shape
    hb = hn.astype(jnp.bfloat16)
    acc = None
    for h in range(nh):
        qk = jnp.dot(hb, wqk_ref[h].astype(jnp.bfloat16),
                     preferred_element_type=jnp.float32)
        v_aug = jnp.dot(hb, wv_ref[h].astype(jnp.bfloat16),
                        preferred_element_type=jnp.float32)
        v_aug = v_aug + jnp.concatenate(
            [jnp.zeros((1, dh), jnp.float32), jnp.ones((1, dh), jnp.float32)],
            axis=-1)
        norm = jnp.sqrt(jnp.sum(qk * qk, axis=-1, keepdims=True))
        kb = (qk / jnp.maximum(norm, 1e-13)).astype(jnp.bfloat16)
        qb = qk.astype(jnp.bfloat16)
        vb = v_aug.astype(jnp.bfloat16)
        o_aug = jnp.zeros((t, 2 * dh), jnp.float32)
        for j in range(t // _JT):
            s = jax.lax.dot_general(
                qb, kb[j * _JT:(j + 1) * _JT],
                (((1,), (1,)), ((), ())), preferred_element_type=jnp.float32)
            e = jnp.exp(s.astype(jnp.bfloat16))
            o_aug = o_aug + jnp.dot(e, vb[j * _JT:(j + 1) * _JT],
                                    preferred_element_type=jnp.float32)
        e_diag = jnp.exp(norm)
        num = o_aug[:, :dh] - e_diag * v_aug[:, :dh]
        den = o_aug[:, dh:dh + 1] - e_diag
        o_h = num / den
        part = jnp.dot(o_h.astype(jnp.bfloat16), wout_ref[h].astype(jnp.bfloat16),
                       preferred_element_type=jnp.float32)
        acc = part if acc is None else acc + part
    return acc


def _ff_block(y1, g2_ref, b2_ref, w1_ref, b1_ref, w2_ref, b2b_ref):
    hh = _layernorm_in(y1, g2_ref[...], b2_ref[...])
    hid = jnp.dot(hh.astype(jnp.bfloat16), w1_ref[...].astype(jnp.bfloat16),
                  preferred_element_type=jnp.float32) + b1_ref[...]
    hid = 0.5 * hid * (1.0 + jax.lax.erf(hid * (2.0 ** -0.5)))
    return jnp.dot(hid.astype(jnp.bfloat16), w2_ref[...].astype(jnp.bfloat16),
                   preferred_element_type=jnp.float32) + b2b_ref[...]


def _layer(x1, x2, hn, wqk_h, wv_aug, wout_h, bout, g2, b2, w1, bias1, w2,
           bias2, gn, bn):
    bb, t, dim = x1.shape
    nh, _, dh2 = wv_aug.shape
    dh = dh2 // 2
    ff = w1.shape[1]

    def kern(x1_ref, x2_ref, hn_ref, wqk_ref, wv_ref, wout_ref, bout_ref,
             g2_ref, b2_ref, w1_ref, b1_ref, w2_ref, b2b_ref, gn_ref, bn_ref,
             y1_ref, y2_ref, hn_out_ref):
        a = _attention_block(hn_ref[0], wqk_ref, wv_ref, wout_ref, nh, dh)
        y1 = x1_ref[0] + a + bout_ref[...]
        y2 = x2_ref[0] + _ff_block(y1, g2_ref, b2_ref, w1_ref, b1_ref,
                                   w2_ref, b2b_ref)
        y1_ref[0] = y1
        y2_ref[0] = y2
        hn_out_ref[0] = _layernorm_in(y2, gn_ref[...], bn_ref[...])

    vec = lambda: pl.BlockSpec((1, dim), lambda i: (0, 0))
    return pl.pallas_call(
        kern,
        grid=(bb,),
        in_specs=[
            pl.BlockSpec((1, t, dim), lambda i: (i, 0, 0)),
            pl.BlockSpec((1, t, dim), lambda i: (i, 0, 0)),
            pl.BlockSpec((1, t, dim), lambda i: (i, 0, 0)),
            pl.BlockSpec((nh, dim, dh), lambda i: (0, 0, 0)),
            pl.BlockSpec((nh, dim, 2 * dh), lambda i: (0, 0, 0)),
            pl.BlockSpec((nh, dh, dim), lambda i: (0, 0, 0)),
            vec(), vec(), vec(),
            pl.BlockSpec((dim, ff), lambda i: (0, 0)),
            pl.BlockSpec((1, ff), lambda i: (0, 0)),
            pl.BlockSpec((ff, dim), lambda i: (0, 0)),
            vec(), vec(), vec(),
        ],
        out_specs=[
            pl.BlockSpec((1, t, dim), lambda i: (i, 0, 0)),
            pl.BlockSpec((1, t, dim), lambda i: (i, 0, 0)),
            pl.BlockSpec((1, t, dim), lambda i: (i, 0, 0)),
        ],
        out_shape=[
            jax.ShapeDtypeStruct((bb, t, dim), jnp.float32),
            jax.ShapeDtypeStruct((bb, t, dim), jnp.float32),
            jax.ShapeDtypeStruct((bb, t, dim), jnp.float32),
        ],
        compiler_params=pltpu.CompilerParams(
            dimension_semantics=("arbitrary",)),
    )(x1, x2, hn, wqk_h, wv_aug, wout_h, bout, g2, b2, w1, bias1, w2, bias2,
      gn, bn)


def _layer_last(x1, x2, hn, wqk_h, wv_aug, wout_h, bout, g2, b2, w1, bias1,
                w2, bias2, nf_g, nf_b, wf, bfv, wc, bcv):
    bb, t, dim = x1.shape
    nh, _, dh2 = wv_aug.shape
    dh = dh2 // 2
    ff = w1.shape[1]
    hid_d = wf.shape[1]
    nc = wc.shape[1]

    def kern(x1_ref, x2_ref, hn_ref, wqk_ref, wv_ref, wout_ref, bout_ref,
             g2_ref, b2_ref, w1_ref, b1_ref, w2_ref, b2b_ref, nfg_ref,
             nfb_ref, wf_ref, bf_ref, wc_ref, bc_ref, o_ref):
        a = _attention_block(hn_ref[0], wqk_ref, wv_ref, wout_ref, nh, dh)
        y1 = x1_ref[0] + a + bout_ref[...]
        y2 = x2_ref[0] + _ff_block(y1, g2_ref, b2_ref, w1_ref, b1_ref,
                                   w2_ref, b2b_ref)
        h = (y1 + y2) * 0.5
        h = _layernorm_in(h, nfg_ref[...], nfb_ref[...])
        hm = jnp.mean(h, axis=0, keepdims=True)
        f = jnp.maximum(jnp.dot(hm, wf_ref[...],
                                preferred_element_type=jnp.float32)
                        + bf_ref[...], 0.0)
        o_ref[0] = jnp.dot(f, wc_ref[...],
                           preferred_element_type=jnp.float32) + bc_ref[...]

    vec = lambda: pl.BlockSpec((1, dim), lambda i: (0, 0))
    return pl.pallas_call(
        kern,
        grid=(bb,),
        in_specs=[
            pl.BlockSpec((1, t, dim), lambda i: (i, 0, 0)),
            pl.BlockSpec((1, t, dim), lambda i: (i, 0, 0)),
            pl.BlockSpec((1, t, dim), lambda i: (i, 0, 0)),
            pl.BlockSpec((nh, dim, dh), lambda i: (0, 0, 0)),
            pl.BlockSpec((nh, dim, 2 * dh), lambda i: (0, 0, 0)),
            pl.BlockSpec((nh, dh, dim), lambda i: (0, 0, 0)),
            vec(), vec(), vec(),
            pl.BlockSpec((dim, ff), lambda i: (0, 0)),
            pl.BlockSpec((1, ff), lambda i: (0, 0)),
            pl.BlockSpec((ff, dim), lambda i: (0, 0)),
            vec(), vec(), vec(),
            pl.BlockSpec((dim, hid_d), lambda i: (0, 0)),
            pl.BlockSpec((1, hid_d), lambda i: (0, 0)),
            pl.BlockSpec((hid_d, nc), lambda i: (0, 0)),
            pl.BlockSpec((1, nc), lambda i: (0, 0)),
        ],
        out_specs=pl.BlockSpec((1, 1, nc), lambda i: (i, 0, 0)),
        out_shape=jax.ShapeDtypeStruct((bb, 1, nc), jnp.float32),
        compiler_params=pltpu.CompilerParams(
            dimension_semantics=("arbitrary",)),
    )(x1, x2, hn, wqk_h, wv_aug, wout_h, bout, g2, b2, w1, bias1, w2, bias2,
      nf_g, nf_b, wf, bfv, wc, bcv)


def kernel(x, emb, pos, ln1_g, ln1_b, Wqk, Wv, Wout, bout, ln2_g, ln2_b,
           W1, b1, W2, b2, nf_g, nf_b, Wf, bf, Wc, bc):
    bb, t = x.shape
    dim = emb.shape[1]
    ll, _, hdh = Wqk.shape
    dh = 64
    nh = hdh // dh

    idx = x.reshape(bb * t).astype(jnp.int32)
    g = _sc_gather(emb, idx).reshape(bb, t, dim)
    h, hn = _embed(g, pos[:t], ln1_g[0].reshape(1, dim),
                   ln1_b[0].reshape(1, dim))

    wqk_h = (Wqk * (dh ** -0.5)).reshape(ll, dim, nh, dh).transpose(0, 2, 1, 3)
    wv_h = Wv.reshape(ll, dim, nh, dh).transpose(0, 2, 1, 3)
    wv_aug = jnp.concatenate([wv_h, jnp.zeros_like(wv_h)], axis=-1)
    wout_h = Wout.reshape(ll, nh, dh, dim)

    x1, x2 = h, h
    for l in range(ll - 1):
        x1, x2, hn = _layer(
            x1, x2, hn, wqk_h[l], wv_aug[l], wout_h[l],
            bout[l].reshape(1, dim), ln2_g[l].reshape(1, dim),
            ln2_b[l].reshape(1, dim), W1[l], b1[l].reshape(1, -1), W2[l],
            b2[l].reshape(1, dim), ln1_g[l + 1].reshape(1, dim),
            ln1_b[l + 1].reshape(1, dim))

    return _layer_last(
        x1, x2, hn, wqk_h[ll - 1], wv_aug[ll - 1], wout_h[ll - 1],
        bout[ll - 1].reshape(1, dim), ln2_g[ll - 1].reshape(1, dim),
        ln2_b[ll - 1].reshape(1, dim), W1[ll - 1], b1[ll - 1].reshape(1, -1),
        W2[ll - 1], b2[ll - 1].reshape(1, dim), nf_g.reshape(1, dim),
        nf_b.reshape(1, dim), Wf, bf.reshape(1, -1), Wc,
        bc.reshape(1, -1)).reshape(bb, -1)


# speedup vs baseline: 7.3994x; 1.3175x over previous
import jax
import jax.numpy as jnp
from jax.experimental import pallas as pl
from jax.experimental.pallas import tpu as pltpu
from jax.experimental.pallas import tpu_sc as plsc


def _sc_gather(emb, idx_flat):
    n = idx_flat.shape[0]
    dim = emb.shape[1]
    window = 128
    indices = idx_flat.reshape(1, n)
    mesh = plsc.VectorSubcoreMesh(core_axis_name="c", subcore_axis_name="s")

    @pl.kernel(out_type=jax.ShapeDtypeStruct((n, dim), emb.dtype), mesh=mesh)
    def gather_kernel(emb_hbm, i_hbm, o_hbm):
        def body(i_vmem, o_vmem):
            pltpu.sync_copy(emb_hbm.at[i_vmem.at[0]], o_vmem)

        pltpu.emit_pipeline(
            body,
            grid=(n // window,),
            in_specs=[pl.BlockSpec((1, window), lambda i: (0, i))],
            out_specs=[pl.BlockSpec((window, dim), lambda i: (i, 0))],
            core_axis_name=("c", "s"),
            dimension_semantics=(pltpu.PARALLEL,),
        )(i_hbm, o_hbm)

    return gather_kernel(emb, indices)


def _embed(g, pos, g1, b1):
    bb, t, dim = g.shape

    def kern(g_ref, p_ref, g1_ref, b1_ref, h_ref, hn_ref):
        h = g_ref[0] + p_ref[...]
        h_ref[0] = h
        hn_ref[0] = _layernorm_in(h, g1_ref[...], b1_ref[...])

    return pl.pallas_call(
        kern,
        grid=(bb,),
        in_specs=[
            pl.BlockSpec((1, t, dim), lambda i: (i, 0, 0)),
            pl.BlockSpec((t, dim), lambda i: (0, 0)),
            pl.BlockSpec((1, dim), lambda i: (0, 0)),
            pl.BlockSpec((1, dim), lambda i: (0, 0)),
        ],
        out_specs=[
            pl.BlockSpec((1, t, dim), lambda i: (i, 0, 0)),
            pl.BlockSpec((1, t, dim), lambda i: (i, 0, 0)),
        ],
        out_shape=[
            jax.ShapeDtypeStruct((bb, t, dim), jnp.float32),
            jax.ShapeDtypeStruct((bb, t, dim), jnp.float32),
        ],
        compiler_params=pltpu.CompilerParams(
            dimension_semantics=("parallel",)),
    )(g, pos, g1, b1)


def _layernorm_in(x, g, b):
    mu = jnp.mean(x, axis=-1, keepdims=True)
    var = jnp.mean((x - mu) ** 2, axis=-1, keepdims=True)
    return (x - mu) * jax.lax.rsqrt(var + 1e-5) * g + b


def _attn_heads(hn, wqk_h, wv_aug):
    bb, t, dim = hn.shape
    nh, _, dh2 = wv_aug.shape
    dh = dh2 // 2

    def kern(x_ref, wqk_ref, wv_ref, o_ref):
        hb = x_ref[0].astype(jnp.bfloat16)
        qk = jnp.dot(hb, wqk_ref[0].astype(jnp.bfloat16),
                     preferred_element_type=jnp.float32)
        v_aug = jnp.dot(hb, wv_ref[0].astype(jnp.bfloat16),
                        preferred_element_type=jnp.float32)
        v_aug = v_aug + jnp.concatenate(
            [jnp.zeros((1, dh), jnp.float32), jnp.ones((1, dh), jnp.float32)],
            axis=-1)
        norm = jnp.sqrt(jnp.sum(qk * qk, axis=-1, keepdims=True))
        kb = (qk / jnp.maximum(norm, 1e-13)).astype(jnp.bfloat16)
        s = jax.lax.dot_general(
            qk.astype(jnp.bfloat16), kb,
            (((1,), (1,)), ((), ())), preferred_element_type=jnp.float32)
        e = jnp.exp(s.astype(jnp.bfloat16))
        o_aug = jnp.dot(e, v_aug.astype(jnp.bfloat16),
                        preferred_element_type=jnp.float32)
        e_diag = jnp.exp(norm)
        num = o_aug[:, :dh] - e_diag * v_aug[:, :dh]
        den = o_aug[:, dh:dh + 1] - e_diag
        o_ref[0, 0] = num / den

    return pl.pallas_call(
        kern,
        grid=(bb, nh),
        in_specs=[
            pl.BlockSpec((1, t, dim), lambda i, j: (i, 0, 0)),
            pl.BlockSpec((1, dim, dh), lambda i, j: (j, 0, 0)),
            pl.BlockSpec((1, dim, 2 * dh), lambda i, j: (j, 0, 0)),
        ],
        out_specs=pl.BlockSpec((1, 1, t, dh), lambda i, j: (i, j, 0, 0)),
        out_shape=jax.ShapeDtypeStruct((bb, nh, t, dh), jnp.float32),
        compiler_params=pltpu.CompilerParams(
            dimension_semantics=("parallel", "parallel")),
    )(hn, wqk_h, wv_aug)


def _merge(x1, x2, o, wout_h, bout, g2, b2, w1, bias1, w2, bias2,
           gn, bn, tblk, emit_ln):
    bb, t, dim = x1.shape
    nh, dh, _ = wout_h.shape
    ff = w1.shape[1]

    def kern(x1_ref, x2_ref, o_ref, wout_ref, bout_ref, g2_ref, b2_ref,
             w1_ref, b1_ref, w2_ref, b2b_ref, gn_ref, bn_ref, *out_refs):
        acc = jnp.zeros((tblk, dim), jnp.float32)
        for h in range(nh):
            acc = acc + jnp.dot(o_ref[0, h].astype(jnp.bfloat16),
                                wout_ref[h].astype(jnp.bfloat16),
                                preferred_element_type=jnp.float32)
        y1 = x1_ref[0] + acc + bout_ref[...]
        hh = _layernorm_in(y1, g2_ref[...], b2_ref[...])
        hid = jnp.dot(hh.astype(jnp.bfloat16), w1_ref[...].astype(jnp.bfloat16),
                      preferred_element_type=jnp.float32) + b1_ref[...]
        hid = 0.5 * hid * (1.0 + jax.lax.erf(hid * (2.0 ** -0.5)))
        y2 = x2_ref[0] + jnp.dot(hid.astype(jnp.bfloat16),
                                 w2_ref[...].astype(jnp.bfloat16),
                                 preferred_element_type=jnp.float32) + b2b_ref[...]
        out_refs[0][0] = y1
        out_refs[1][0] = y2
        if emit_ln:
            out_refs[2][0] = _layernorm_in(y2, gn_ref[...], bn_ref[...])

    nblk = t // tblk
    n_out = 3 if emit_ln else 2
    blk3 = lambda: pl.BlockSpec((1, tblk, dim), lambda i, j: (i, j, 0))
    return pl.pallas_call(
        kern,
        grid=(bb, nblk),
        in_specs=[
            blk3(),
            blk3(),
            pl.BlockSpec((1, nh, tblk, dh), lambda i, j: (i, 0, j, 0)),
            pl.BlockSpec((nh, dh, dim), lambda i, j: (0, 0, 0)),
            pl.BlockSpec((1, dim), lambda i, j: (0, 0)),
            pl.BlockSpec((1, dim), lambda i, j: (0, 0)),
            pl.BlockSpec((1, dim), lambda i, j: (0, 0)),
            pl.BlockSpec((dim, ff), lambda i, j: (0, 0)),
            pl.BlockSpec((1, ff), lambda i, j: (0, 0)),
            pl.BlockSpec((ff, dim), lambda i, j: (0, 0)),
            pl.BlockSpec((1, dim), lambda i, j: (0, 0)),
            pl.BlockSpec((1, dim), lambda i, j: (0, 0)),
            pl.BlockSpec((1, dim), lambda i, j: (0, 0)),
        ],
        out_specs=[blk3() for _ in range(n_out)],
        out_shape=[jax.ShapeDtypeStruct((bb, t, dim), jnp.float32)
                   for _ in range(n_out)],
        compiler_params=pltpu.CompilerParams(
            dimension_semantics=("parallel", "parallel")),
    )(x1, x2, o, wout_h, bout, g2, b2, w1, bias1, w2, bias2, gn, bn)


def _head(x1, x2, nf_g, nf_b, wf, bf, wc, bc):
    bb, t, dim = x1.shape
    hid = wf.shape[1]
    nc = wc.shape[1]

    def kern(x1_ref, x2_ref, g_ref, b_ref, wf_ref, bf_ref, wc_ref, bc_ref,
             o_ref):
        h = (x1_ref[...] + x2_ref[...]) * 0.5
        h = _layernorm_in(h, g_ref[...], b_ref[...])
        hm = jnp.mean(h, axis=1)
        f = jnp.maximum(jnp.dot(hm, wf_ref[...],
                                preferred_element_type=jnp.float32)
                        + bf_ref[...], 0.0)
        o_ref[...] = jnp.dot(f, wc_ref[...],
                             preferred_element_type=jnp.float32) + bc_ref[...]

    return pl.pallas_call(
        kern,
        in_specs=[
            pl.BlockSpec((bb, t, dim), lambda: (0, 0, 0)),
            pl.BlockSpec((bb, t, dim), lambda: (0, 0, 0)),
            pl.BlockSpec((1, dim), lambda: (0, 0)),
            pl.BlockSpec((1, dim), lambda: (0, 0)),
            pl.BlockSpec((dim, hid), lambda: (0, 0)),
            pl.BlockSpec((1, hid), lambda: (0, 0)),
            pl.BlockSpec((hid, nc), lambda: (0, 0)),
            pl.BlockSpec((1, nc), lambda: (0, 0)),
        ],
        out_specs=pl.BlockSpec((bb, nc), lambda: (0, 0)),
        out_shape=jax.ShapeDtypeStruct((bb, nc), jnp.float32),
    )(x1, x2, nf_g, nf_b, wf, bf, wc, bc)


def kernel(x, emb, pos, ln1_g, ln1_b, Wqk, Wv, Wout, bout, ln2_g, ln2_b,
           W1, b1, W2, b2, nf_g, nf_b, Wf, bf, Wc, bc):
    bb, t = x.shape
    dim = emb.shape[1]
    ll, _, hdh = Wqk.shape
    dh = 64
    nh = hdh // dh

    idx = x.reshape(bb * t).astype(jnp.int32)
    g = _sc_gather(emb, idx).reshape(bb, t, dim)
    h, hn = _embed(g, pos[:t], ln1_g[0].reshape(1, dim),
                   ln1_b[0].reshape(1, dim))

    wqk_h = (Wqk * (dh ** -0.5)).reshape(ll, dim, nh, dh).transpose(0, 2, 1, 3)
    wv_h = Wv.reshape(ll, dim, nh, dh).transpose(0, 2, 1, 3)
    wv_aug = jnp.concatenate([wv_h, jnp.zeros_like(wv_h)], axis=-1)
    wout_h = Wout.reshape(ll, nh, dh, dim)

    x1, x2 = h, h
    for l in range(ll):
        o = _attn_heads(hn, wqk_h[l], wv_aug[l])
        last = l == ll - 1
        gn = ln1_g[0 if last else l + 1].reshape(1, dim)
        bn = ln1_b[0 if last else l + 1].reshape(1, dim)
        outs = _merge(x1, x2, o, wout_h[l], bout[l].reshape(1, dim),
                      ln2_g[l].reshape(1, dim), ln2_b[l].reshape(1, dim),
                      W1[l], b1[l].reshape(1, -1), W2[l],
                      b2[l].reshape(1, dim), gn, bn, tblk=512,
                      emit_ln=not last)
        x1, x2 = outs[0], outs[1]
        if not last:
            hn = outs[2]

    return _head(x1, x2, nf_g.reshape(1, dim), nf_b.reshape(1, dim),
                 Wf, bf.reshape(1, -1), Wc, bc.reshape(1, -1))


# speedup vs baseline: 7.6057x; 1.0279x over previous
import jax
import jax.numpy as jnp
from jax.experimental import pallas as pl
from jax.experimental.pallas import tpu as pltpu
from jax.experimental.pallas import tpu_sc as plsc


def _sc_gather(emb, idx_flat):
    n = idx_flat.shape[0]
    dim = emb.shape[1]
    window = 128
    indices = idx_flat.reshape(1, n)
    mesh = plsc.VectorSubcoreMesh(core_axis_name="c", subcore_axis_name="s")

    @pl.kernel(out_type=jax.ShapeDtypeStruct((n, dim), emb.dtype), mesh=mesh)
    def gather_kernel(emb_hbm, i_hbm, o_hbm):
        def body(i_vmem, o_vmem):
            pltpu.sync_copy(emb_hbm.at[i_vmem.at[0]], o_vmem)

        pltpu.emit_pipeline(
            body,
            grid=(n // window,),
            in_specs=[pl.BlockSpec((1, window), lambda i: (0, i))],
            out_specs=[pl.BlockSpec((window, dim), lambda i: (i, 0))],
            core_axis_name=("c", "s"),
            dimension_semantics=(pltpu.PARALLEL,),
        )(i_hbm, o_hbm)

    return gather_kernel(emb, indices)


def _embed(g, pos, g1, b1):
    bb, t, dim = g.shape

    def kern(g_ref, p_ref, g1_ref, b1_ref, h_ref, hn_ref):
        h = g_ref[0] + p_ref[...]
        h_ref[0] = h
        hn_ref[0] = _layernorm_in(h, g1_ref[...], b1_ref[...])

    return pl.pallas_call(
        kern,
        grid=(bb,),
        in_specs=[
            pl.BlockSpec((1, t, dim), lambda i: (i, 0, 0)),
            pl.BlockSpec((t, dim), lambda i: (0, 0)),
            pl.BlockSpec((1, dim), lambda i: (0, 0)),
            pl.BlockSpec((1, dim), lambda i: (0, 0)),
        ],
        out_specs=[
            pl.BlockSpec((1, t, dim), lambda i: (i, 0, 0)),
            pl.BlockSpec((1, t, dim), lambda i: (i, 0, 0)),
        ],
        out_shape=[
            jax.ShapeDtypeStruct((bb, t, dim), jnp.float32),
            jax.ShapeDtypeStruct((bb, t, dim), jnp.float32),
        ],
        compiler_params=pltpu.CompilerParams(
            dimension_semantics=("parallel",)),
    )(g, pos, g1, b1)


def _layernorm_in(x, g, b):
    mu = jnp.mean(x, axis=-1, keepdims=True)
    var = jnp.mean((x - mu) ** 2, axis=-1, keepdims=True)
    return (x - mu) * jax.lax.rsqrt(var + 1e-5) * g + b


def _attn_heads(hn, wqk_h, wv_aug):
    bb, t, dim = hn.shape
    nh, _, dh2 = wv_aug.shape
    dh = dh2 // 2

    def kern(x_ref, wqk_ref, wv_ref, o_ref):
        hb = x_ref[0].astype(jnp.bfloat16)
        qk = jnp.dot(hb, wqk_ref[0].astype(jnp.bfloat16),
                     preferred_element_type=jnp.float32)
        v_aug = jnp.dot(hb, wv_ref[0].astype(jnp.bfloat16),
                        preferred_element_type=jnp.float32)
        v_aug = v_aug + jnp.concatenate(
            [jnp.zeros((1, dh), jnp.float32), jnp.ones((1, dh), jnp.float32)],
            axis=-1)
        norm = jnp.sqrt(jnp.sum(qk * qk, axis=-1, keepdims=True))
        kb = (qk / jnp.maximum(norm, 1e-13)).astype(jnp.bfloat16)
        s = jax.lax.dot_general(
            qk.astype(jnp.bfloat16), kb,
            (((1,), (1,)), ((), ())), preferred_element_type=jnp.float32)
        e = jnp.exp(s.astype(jnp.bfloat16))
        o_aug = jnp.dot(e, v_aug.astype(jnp.bfloat16),
                        preferred_element_type=jnp.float32)
        e_diag = jnp.exp(norm)
        num = o_aug[:, :dh] - e_diag * v_aug[:, :dh]
        den = o_aug[:, dh:dh + 1] - e_diag
        o_ref[0, 0] = num / den

    return pl.pallas_call(
        kern,
        grid=(bb, nh),
        in_specs=[
            pl.BlockSpec((1, t, dim), lambda i, j: (i, 0, 0)),
            pl.BlockSpec((1, dim, dh), lambda i, j: (j, 0, 0)),
            pl.BlockSpec((1, dim, 2 * dh), lambda i, j: (j, 0, 0)),
        ],
        out_specs=pl.BlockSpec((1, 1, t, dh), lambda i, j: (i, j, 0, 0)),
        out_shape=jax.ShapeDtypeStruct((bb, nh, t, dh), jnp.float32),
        compiler_params=pltpu.CompilerParams(
            dimension_semantics=("parallel", "parallel")),
    )(hn, wqk_h, wv_aug)


def _merge(x1, x2, o, wout_h, bout, g2, b2, w1, bias1, w2, bias2,
           gn, bn, tblk, emit_ln):
    bb, t, dim = x1.shape
    nh, dh, _ = wout_h.shape
    ff = w1.shape[1]

    def kern(x1_ref, x2_ref, o_ref, wout_ref, bout_ref, g2_ref, b2_ref,
             w1_ref, b1_ref, w2_ref, b2b_ref, gn_ref, bn_ref, *out_refs):
        acc = jnp.zeros((tblk, dim), jnp.float32)
        for h in range(nh):
            acc = acc + jnp.dot(o_ref[0, h].astype(jnp.bfloat16),
                                wout_ref[h].astype(jnp.bfloat16),
                                preferred_element_type=jnp.float32)
        y1 = x1_ref[0] + acc + bout_ref[...]
        hh = _layernorm_in(y1, g2_ref[...], b2_ref[...])
        hid = jnp.dot(hh.astype(jnp.bfloat16), w1_ref[...].astype(jnp.bfloat16),
                      preferred_element_type=jnp.float32) + b1_ref[...]
        hid = 0.5 * hid * (1.0 + jax.lax.erf(hid * (2.0 ** -0.5)))
        y2 = x2_ref[0] + jnp.dot(hid.astype(jnp.bfloat16),
                                 w2_ref[...].astype(jnp.bfloat16),
                                 preferred_element_type=jnp.float32) + b2b_ref[...]
        out_refs[0][0] = y1
        out_refs[1][0] = y2
        if emit_ln:
            out_refs[2][0] = _layernorm_in(y2, gn_ref[...], bn_ref[...])

    nblk = t // tblk
    n_out = 3 if emit_ln else 2
    blk3 = lambda: pl.BlockSpec((1, tblk, dim), lambda i, j: (i, j, 0))
    return pl.pallas_call(
        kern,
        grid=(bb, nblk),
        in_specs=[
            blk3(),
            blk3(),
            pl.BlockSpec((1, nh, tblk, dh), lambda i, j: (i, 0, j, 0)),
            pl.BlockSpec((nh, dh, dim), lambda i, j: (0, 0, 0)),
            pl.BlockSpec((1, dim), lambda i, j: (0, 0)),
            pl.BlockSpec((1, dim), lambda i, j: (0, 0)),
            pl.BlockSpec((1, dim), lambda i, j: (0, 0)),
            pl.BlockSpec((dim, ff), lambda i, j: (0, 0)),
            pl.BlockSpec((1, ff), lambda i, j: (0, 0)),
            pl.BlockSpec((ff, dim), lambda i, j: (0, 0)),
            pl.BlockSpec((1, dim), lambda i, j: (0, 0)),
            pl.BlockSpec((1, dim), lambda i, j: (0, 0)),
            pl.BlockSpec((1, dim), lambda i, j: (0, 0)),
        ],
        out_specs=[blk3() for _ in range(n_out)],
        out_shape=[jax.ShapeDtypeStruct((bb, t, dim), jnp.float32)
                   for _ in range(n_out)],
        compiler_params=pltpu.CompilerParams(
            dimension_semantics=("parallel", "parallel")),
    )(x1, x2, o, wout_h, bout, g2, b2, w1, bias1, w2, bias2, gn, bn)


def _head(x1, x2, nf_g, nf_b, wf, bf, wc, bc):
    bb, t, dim = x1.shape
    hid = wf.shape[1]
    nc = wc.shape[1]

    def kern(x1_ref, x2_ref, g_ref, b_ref, wf_ref, bf_ref, wc_ref, bc_ref,
             o_ref):
        h = (x1_ref[...] + x2_ref[...]) * 0.5
        h = _layernorm_in(h, g_ref[...], b_ref[...])
        hm = jnp.mean(h, axis=1)
        f = jnp.maximum(jnp.dot(hm, wf_ref[...],
                                preferred_element_type=jnp.float32)
                        + bf_ref[...], 0.0)
        o_ref[...] = jnp.dot(f, wc_ref[...],
                             preferred_element_type=jnp.float32) + bc_ref[...]

    return pl.pallas_call(
        kern,
        in_specs=[
            pl.BlockSpec((bb, t, dim), lambda: (0, 0, 0)),
            pl.BlockSpec((bb, t, dim), lambda: (0, 0, 0)),
            pl.BlockSpec((1, dim), lambda: (0, 0)),
            pl.BlockSpec((1, dim), lambda: (0, 0)),
            pl.BlockSpec((dim, hid), lambda: (0, 0)),
            pl.BlockSpec((1, hid), lambda: (0, 0)),
            pl.BlockSpec((hid, nc), lambda: (0, 0)),
            pl.BlockSpec((1, nc), lambda: (0, 0)),
        ],
        out_specs=pl.BlockSpec((bb, nc), lambda: (0, 0)),
        out_shape=jax.ShapeDtypeStruct((bb, nc), jnp.float32),
    )(x1, x2, nf_g, nf_b, wf, bf, wc, bc)


def kernel(x, emb, pos, ln1_g, ln1_b, Wqk, Wv, Wout, bout, ln2_g, ln2_b,
           W1, b1, W2, b2, nf_g, nf_b, Wf, bf, Wc, bc):
    bb, t = x.shape
    dim = emb.shape[1]
    ll, _, hdh = Wqk.shape
    dh = 64
    nh = hdh // dh

    idx = x.reshape(bb * t).astype(jnp.int32)
    g = _sc_gather(emb, idx).reshape(bb, t, dim)
    h, hn = _embed(g, pos[:t], ln1_g[0].reshape(1, dim),
                   ln1_b[0].reshape(1, dim))

    wqk_h = (Wqk * (dh ** -0.5)).reshape(ll, dim, nh, dh).transpose(0, 2, 1, 3)
    wv_h = Wv.reshape(ll, dim, nh, dh).transpose(0, 2, 1, 3)
    wv_aug = jnp.concatenate([wv_h, jnp.zeros_like(wv_h)], axis=-1)
    wout_h = Wout.reshape(ll, nh, dh, dim)

    x1, x2 = h, h
    for l in range(ll):
        o = _attn_heads(hn, wqk_h[l], wv_aug[l])
        last = l == ll - 1
        gn = ln1_g[0 if last else l + 1].reshape(1, dim)
        bn = ln1_b[0 if last else l + 1].reshape(1, dim)
        outs = _merge(x1, x2, o, wout_h[l], bout[l].reshape(1, dim),
                      ln2_g[l].reshape(1, dim), ln2_b[l].reshape(1, dim),
                      W1[l], b1[l].reshape(1, -1), W2[l],
                      b2[l].reshape(1, dim), gn, bn, tblk=1024,
                      emit_ln=not last)
        x1, x2 = outs[0], outs[1]
        if not last:
            hn = outs[2]

    return _head(x1, x2, nf_g.reshape(1, dim), nf_b.reshape(1, dim),
                 Wf, bf.reshape(1, -1), Wc, bc.reshape(1, -1))


# speedup vs baseline: 7.7917x; 1.0245x over previous
import jax
import jax.numpy as jnp
from jax.experimental import pallas as pl
from jax.experimental.pallas import tpu as pltpu
from jax.experimental.pallas import tpu_sc as plsc


def _sc_gather(emb, idx_flat):
    n = idx_flat.shape[0]
    dim = emb.shape[1]
    window = 128
    indices = idx_flat.reshape(1, n)
    mesh = plsc.VectorSubcoreMesh(core_axis_name="c", subcore_axis_name="s")

    @pl.kernel(out_type=jax.ShapeDtypeStruct((n, dim), emb.dtype), mesh=mesh)
    def gather_kernel(emb_hbm, i_hbm, o_hbm):
        def body(i_vmem, o_vmem):
            pltpu.sync_copy(emb_hbm.at[i_vmem.at[0]], o_vmem)

        pltpu.emit_pipeline(
            body,
            grid=(n // window,),
            in_specs=[pl.BlockSpec((1, window), lambda i: (0, i))],
            out_specs=[pl.BlockSpec((window, dim), lambda i: (i, 0))],
            core_axis_name=("c", "s"),
            dimension_semantics=(pltpu.PARALLEL,),
        )(i_hbm, o_hbm)

    return gather_kernel(emb, indices)


def _embed(g, pos, g1, b1):
    bb, t, dim = g.shape

    def kern(g_ref, p_ref, g1_ref, b1_ref, h_ref, hn_ref):
        h = g_ref[0] + p_ref[...]
        h_ref[0] = h
        hn_ref[0] = _layernorm_in(h, g1_ref[...], b1_ref[...])

    return pl.pallas_call(
        kern,
        grid=(bb,),
        in_specs=[
            pl.BlockSpec((1, t, dim), lambda i: (i, 0, 0)),
            pl.BlockSpec((t, dim), lambda i: (0, 0)),
            pl.BlockSpec((1, dim), lambda i: (0, 0)),
            pl.BlockSpec((1, dim), lambda i: (0, 0)),
        ],
        out_specs=[
            pl.BlockSpec((1, t, dim), lambda i: (i, 0, 0)),
            pl.BlockSpec((1, t, dim), lambda i: (i, 0, 0)),
        ],
        out_shape=[
            jax.ShapeDtypeStruct((bb, t, dim), jnp.float32),
            jax.ShapeDtypeStruct((bb, t, dim), jnp.float32),
        ],
        compiler_params=pltpu.CompilerParams(
            dimension_semantics=("parallel",)),
    )(g, pos, g1, b1)


def _layernorm_in(x, g, b):
    mu = jnp.mean(x, axis=-1, keepdims=True)
    var = jnp.mean((x - mu) ** 2, axis=-1, keepdims=True)
    return (x - mu) * jax.lax.rsqrt(var + 1e-5) * g + b


def _attn_heads(hn, wqk_h, wv_aug):
    bb, t, dim = hn.shape
    nh, _, dh2 = wv_aug.shape
    dh = dh2 // 2

    def kern(x_ref, wqk_ref, wv_ref, o_ref):
        for b in range(bb):
            hb = x_ref[b].astype(jnp.bfloat16)
            qk = jnp.dot(hb, wqk_ref[0].astype(jnp.bfloat16),
                         preferred_element_type=jnp.float32)
            v_aug = jnp.dot(hb, wv_ref[0].astype(jnp.bfloat16),
                            preferred_element_type=jnp.float32)
            v_aug = v_aug + jnp.concatenate(
                [jnp.zeros((1, dh), jnp.float32),
                 jnp.ones((1, dh), jnp.float32)], axis=-1)
            norm = jnp.sqrt(jnp.sum(qk * qk, axis=-1, keepdims=True))
            kb = (qk / jnp.maximum(norm, 1e-13)).astype(jnp.bfloat16)
            s = jax.lax.dot_general(
                qk.astype(jnp.bfloat16), kb,
                (((1,), (1,)), ((), ())), preferred_element_type=jnp.float32)
            e = jnp.exp(s.astype(jnp.bfloat16))
            o_aug = jnp.dot(e, v_aug.astype(jnp.bfloat16),
                            preferred_element_type=jnp.float32)
            e_diag = jnp.exp(norm)
            num = o_aug[:, :dh] - e_diag * v_aug[:, :dh]
            den = o_aug[:, dh:dh + 1] - e_diag
            o_ref[b, 0] = num / den

    return pl.pallas_call(
        kern,
        grid=(nh,),
        in_specs=[
            pl.BlockSpec((bb, t, dim), lambda j: (0, 0, 0)),
            pl.BlockSpec((1, dim, dh), lambda j: (j, 0, 0)),
            pl.BlockSpec((1, dim, 2 * dh), lambda j: (j, 0, 0)),
        ],
        out_specs=pl.BlockSpec((bb, 1, t, dh), lambda j: (0, j, 0, 0)),
        out_shape=jax.ShapeDtypeStruct((bb, nh, t, dh), jnp.float32),
        compiler_params=pltpu.CompilerParams(
            dimension_semantics=("parallel",)),
    )(hn, wqk_h, wv_aug)


def _merge(x1, x2, o, wout_h, bout, g2, b2, w1, bias1, w2, bias2,
           gn, bn, tblk, emit_ln):
    bb, t, dim = x1.shape
    nh, dh, _ = wout_h.shape
    ff = w1.shape[1]

    def kern(x1_ref, x2_ref, o_ref, wout_ref, bout_ref, g2_ref, b2_ref,
             w1_ref, b1_ref, w2_ref, b2b_ref, gn_ref, bn_ref, *out_refs):
        acc = jnp.zeros((tblk, dim), jnp.float32)
        for h in range(nh):
            acc = acc + jnp.dot(o_ref[0, h].astype(jnp.bfloat16),
                                wout_ref[h].astype(jnp.bfloat16),
                                preferred_element_type=jnp.float32)
        y1 = x1_ref[0] + acc + bout_ref[...]
        hh = _layernorm_in(y1, g2_ref[...], b2_ref[...])
        hid = jnp.dot(hh.astype(jnp.bfloat16), w1_ref[...].astype(jnp.bfloat16),
                      preferred_element_type=jnp.float32) + b1_ref[...]
        hid = 0.5 * hid * (1.0 + jax.lax.erf(hid * (2.0 ** -0.5)))
        y2 = x2_ref[0] + jnp.dot(hid.astype(jnp.bfloat16),
                                 w2_ref[...].astype(jnp.bfloat16),
                                 preferred_element_type=jnp.float32) + b2b_ref[...]
        out_refs[0][0] = y1
        out_refs[1][0] = y2
        if emit_ln:
            out_refs[2][0] = _layernorm_in(y2, gn_ref[...], bn_ref[...])

    nblk = t // tblk
    n_out = 3 if emit_ln else 2
    blk3 = lambda: pl.BlockSpec((1, tblk, dim), lambda i, j: (i, j, 0))
    return pl.pallas_call(
        kern,
        grid=(bb, nblk),
        in_specs=[
            blk3(),
            blk3(),
            pl.BlockSpec((1, nh, tblk, dh), lambda i, j: (i, 0, j, 0)),
            pl.BlockSpec((nh, dh, dim), lambda i, j: (0, 0, 0)),
            pl.BlockSpec((1, dim), lambda i, j: (0, 0)),
            pl.BlockSpec((1, dim), lambda i, j: (0, 0)),
            pl.BlockSpec((1, dim), lambda i, j: (0, 0)),
            pl.BlockSpec((dim, ff), lambda i, j: (0, 0)),
            pl.BlockSpec((1, ff), lambda i, j: (0, 0)),
            pl.BlockSpec((ff, dim), lambda i, j: (0, 0)),
            pl.BlockSpec((1, dim), lambda i, j: (0, 0)),
            pl.BlockSpec((1, dim), lambda i, j: (0, 0)),
            pl.BlockSpec((1, dim), lambda i, j: (0, 0)),
        ],
        out_specs=[blk3() for _ in range(n_out)],
        out_shape=[jax.ShapeDtypeStruct((bb, t, dim), jnp.float32)
                   for _ in range(n_out)],
        compiler_params=pltpu.CompilerParams(
            dimension_semantics=("parallel", "parallel")),
    )(x1, x2, o, wout_h, bout, g2, b2, w1, bias1, w2, bias2, gn, bn)


def _head(x1, x2, nf_g, nf_b, wf, bf, wc, bc):
    bb, t, dim = x1.shape
    hid = wf.shape[1]
    nc = wc.shape[1]

    def kern(x1_ref, x2_ref, g_ref, b_ref, wf_ref, bf_ref, wc_ref, bc_ref,
             o_ref):
        h = (x1_ref[...] + x2_ref[...]) * 0.5
        h = _layernorm_in(h, g_ref[...], b_ref[...])
        hm = jnp.mean(h, axis=1)
        f = jnp.maximum(jnp.dot(hm, wf_ref[...],
                                preferred_element_type=jnp.float32)
                        + bf_ref[...], 0.0)
        o_ref[...] = jnp.dot(f, wc_ref[...],
                             preferred_element_type=jnp.float32) + bc_ref[...]

    return pl.pallas_call(
        kern,
        in_specs=[
            pl.BlockSpec((bb, t, dim), lambda: (0, 0, 0)),
            pl.BlockSpec((bb, t, dim), lambda: (0, 0, 0)),
            pl.BlockSpec((1, dim), lambda: (0, 0)),
            pl.BlockSpec((1, dim), lambda: (0, 0)),
            pl.BlockSpec((dim, hid), lambda: (0, 0)),
            pl.BlockSpec((1, hid), lambda: (0, 0)),
            pl.BlockSpec((hid, nc), lambda: (0, 0)),
            pl.BlockSpec((1, nc), lambda: (0, 0)),
        ],
        out_specs=pl.BlockSpec((bb, nc), lambda: (0, 0)),
        out_shape=jax.ShapeDtypeStruct((bb, nc), jnp.float32),
    )(x1, x2, nf_g, nf_b, wf, bf, wc, bc)


def kernel(x, emb, pos, ln1_g, ln1_b, Wqk, Wv, Wout, bout, ln2_g, ln2_b,
           W1, b1, W2, b2, nf_g, nf_b, Wf, bf, Wc, bc):
    bb, t = x.shape
    dim = emb.shape[1]
    ll, _, hdh = Wqk.shape
    dh = 64
    nh = hdh // dh

    idx = x.reshape(bb * t).astype(jnp.int32)
    g = _sc_gather(emb, idx).reshape(bb, t, dim)
    h, hn = _embed(g, pos[:t], ln1_g[0].reshape(1, dim),
                   ln1_b[0].reshape(1, dim))

    wqk_h = (Wqk * (dh ** -0.5)).reshape(ll, dim, nh, dh).transpose(0, 2, 1, 3)
    wv_h = Wv.reshape(ll, dim, nh, dh).transpose(0, 2, 1, 3)
    wv_aug = jnp.concatenate([wv_h, jnp.zeros_like(wv_h)], axis=-1)
    wout_h = Wout.reshape(ll, nh, dh, dim)

    x1, x2 = h, h
    for l in range(ll):
        o = _attn_heads(hn, wqk_h[l], wv_aug[l])
        last = l == ll - 1
        gn = ln1_g[0 if last else l + 1].reshape(1, dim)
        bn = ln1_b[0 if last else l + 1].reshape(1, dim)
        outs = _merge(x1, x2, o, wout_h[l], bout[l].reshape(1, dim),
                      ln2_g[l].reshape(1, dim), ln2_b[l].reshape(1, dim),
                      W1[l], b1[l].reshape(1, -1), W2[l],
                      b2[l].reshape(1, dim), gn, bn, tblk=1024,
                      emit_ln=not last)
        x1, x2 = outs[0], outs[1]
        if not last:
            hn = outs[2]

    return _head(x1, x2, nf_g.reshape(1, dim), nf_b.reshape(1, dim),
                 Wf, bf.reshape(1, -1), Wc, bc.reshape(1, -1))


# speedup vs baseline: 7.9765x; 1.0237x over previous
import jax
import jax.numpy as jnp
from jax.experimental import pallas as pl
from jax.experimental.pallas import tpu as pltpu
from jax.experimental.pallas import tpu_sc as plsc


def _sc_gather(emb, idx_flat):
    n = idx_flat.shape[0]
    dim = emb.shape[1]
    window = 128
    indices = idx_flat.reshape(1, n)
    mesh = plsc.VectorSubcoreMesh(core_axis_name="c", subcore_axis_name="s")

    @pl.kernel(out_type=jax.ShapeDtypeStruct((n, dim), emb.dtype), mesh=mesh)
    def gather_kernel(emb_hbm, i_hbm, o_hbm):
        def body(i_vmem, o_vmem):
            pltpu.sync_copy(emb_hbm.at[i_vmem.at[0]], o_vmem)

        pltpu.emit_pipeline(
            body,
            grid=(n // window,),
            in_specs=[pl.BlockSpec((1, window), lambda i: (0, i))],
            out_specs=[pl.BlockSpec((window, dim), lambda i: (i, 0))],
            core_axis_name=("c", "s"),
            dimension_semantics=(pltpu.PARALLEL,),
        )(i_hbm, o_hbm)

    return gather_kernel(emb, indices)


def _embed(g, pos, g1, b1):
    bb, t, dim = g.shape

    def kern(g_ref, p_ref, g1_ref, b1_ref, h_ref, hn_ref):
        h = g_ref[0] + p_ref[...]
        h_ref[0] = h
        hn_ref[0] = _layernorm_in(h, g1_ref[...], b1_ref[...]).astype(
            jnp.bfloat16)

    return pl.pallas_call(
        kern,
        grid=(bb,),
        in_specs=[
            pl.BlockSpec((1, t, dim), lambda i: (i, 0, 0)),
            pl.BlockSpec((t, dim), lambda i: (0, 0)),
            pl.BlockSpec((1, dim), lambda i: (0, 0)),
            pl.BlockSpec((1, dim), lambda i: (0, 0)),
        ],
        out_specs=[
            pl.BlockSpec((1, t, dim), lambda i: (i, 0, 0)),
            pl.BlockSpec((1, t, dim), lambda i: (i, 0, 0)),
        ],
        out_shape=[
            jax.ShapeDtypeStruct((bb, t, dim), jnp.float32),
            jax.ShapeDtypeStruct((bb, t, dim), jnp.bfloat16),
        ],
        compiler_params=pltpu.CompilerParams(
            dimension_semantics=("parallel",)),
    )(g, pos, g1, b1)


def _layernorm_in(x, g, b):
    mu = jnp.mean(x, axis=-1, keepdims=True)
    var = jnp.mean((x - mu) ** 2, axis=-1, keepdims=True)
    return (x - mu) * jax.lax.rsqrt(var + 1e-5) * g + b


def _attn_heads(hn, wqk_h, wv_aug):
    bb, t, dim = hn.shape
    nh, _, dh2 = wv_aug.shape
    dh = dh2 // 2

    def kern(x_ref, wqk_ref, wv_ref, o_ref):
        for b in range(bb):
            hb = x_ref[b]
            qk = jnp.dot(hb, wqk_ref[0].astype(jnp.bfloat16),
                         preferred_element_type=jnp.float32)
            v_aug = jnp.dot(hb, wv_ref[0].astype(jnp.bfloat16),
                            preferred_element_type=jnp.float32)
            v_aug = v_aug + jnp.concatenate(
                [jnp.zeros((1, dh), jnp.float32),
                 jnp.ones((1, dh), jnp.float32)], axis=-1)
            norm = jnp.sqrt(jnp.sum(qk * qk, axis=-1, keepdims=True))
            kb = (qk / jnp.maximum(norm, 1e-13)).astype(jnp.bfloat16)
            s = jax.lax.dot_general(
                qk.astype(jnp.bfloat16), kb,
                (((1,), (1,)), ((), ())), preferred_element_type=jnp.float32)
            e = jnp.exp(s.astype(jnp.bfloat16))
            o_aug = jnp.dot(e, v_aug.astype(jnp.bfloat16),
                            preferred_element_type=jnp.float32)
            e_diag = jnp.exp(norm)
            num = o_aug[:, :dh] - e_diag * v_aug[:, :dh]
            den = o_aug[:, dh:dh + 1] - e_diag
            o_ref[b, 0] = (num / den).astype(jnp.bfloat16)

    return pl.pallas_call(
        kern,
        grid=(nh,),
        in_specs=[
            pl.BlockSpec((bb, t, dim), lambda j: (0, 0, 0)),
            pl.BlockSpec((1, dim, dh), lambda j: (j, 0, 0)),
            pl.BlockSpec((1, dim, 2 * dh), lambda j: (j, 0, 0)),
        ],
        out_specs=pl.BlockSpec((bb, 1, t, dh), lambda j: (0, j, 0, 0)),
        out_shape=jax.ShapeDtypeStruct((bb, nh, t, dh), jnp.bfloat16),
        compiler_params=pltpu.CompilerParams(
            dimension_semantics=("parallel",)),
    )(hn, wqk_h, wv_aug)


def _merge(x1, x2, o, wout_h, bout, g2, b2, w1, bias1, w2, bias2,
           gn, bn, tblk, emit_ln):
    bb, t, dim = x1.shape
    nh, dh, _ = wout_h.shape
    ff = w1.shape[1]

    def kern(x1_ref, x2_ref, o_ref, wout_ref, bout_ref, g2_ref, b2_ref,
             w1_ref, b1_ref, w2_ref, b2b_ref, gn_ref, bn_ref, *out_refs):
        acc = jnp.zeros((tblk, dim), jnp.float32)
        for h in range(nh):
            acc = acc + jnp.dot(o_ref[0, h],
                                wout_ref[h].astype(jnp.bfloat16),
                                preferred_element_type=jnp.float32)
        y1 = x1_ref[0] + acc + bout_ref[...]
        hh = _layernorm_in(y1, g2_ref[...], b2_ref[...])
        hid = jnp.dot(hh.astype(jnp.bfloat16), w1_ref[...].astype(jnp.bfloat16),
                      preferred_element_type=jnp.float32) + b1_ref[...]
        hid = 0.5 * hid * (1.0 + jax.lax.erf(hid * (2.0 ** -0.5)))
        y2 = x2_ref[0] + jnp.dot(hid.astype(jnp.bfloat16),
                                 w2_ref[...].astype(jnp.bfloat16),
                                 preferred_element_type=jnp.float32) + b2b_ref[...]
        out_refs[0][0] = y1
        out_refs[1][0] = y2
        if emit_ln:
            out_refs[2][0] = _layernorm_in(
                y2, gn_ref[...], bn_ref[...]).astype(jnp.bfloat16)

    nblk = t // tblk
    n_out = 3 if emit_ln else 2
    blk3 = lambda: pl.BlockSpec((1, tblk, dim), lambda i, j: (i, j, 0))
    return pl.pallas_call(
        kern,
        grid=(bb, nblk),
        in_specs=[
            blk3(),
            blk3(),
            pl.BlockSpec((1, nh, tblk, dh), lambda i, j: (i, 0, j, 0)),
            pl.BlockSpec((nh, dh, dim), lambda i, j: (0, 0, 0)),
            pl.BlockSpec((1, dim), lambda i, j: (0, 0)),
            pl.BlockSpec((1, dim), lambda i, j: (0, 0)),
            pl.BlockSpec((1, dim), lambda i, j: (0, 0)),
            pl.BlockSpec((dim, ff), lambda i, j: (0, 0)),
            pl.BlockSpec((1, ff), lambda i, j: (0, 0)),
            pl.BlockSpec((ff, dim), lambda i, j: (0, 0)),
            pl.BlockSpec((1, dim), lambda i, j: (0, 0)),
            pl.BlockSpec((1, dim), lambda i, j: (0, 0)),
            pl.BlockSpec((1, dim), lambda i, j: (0, 0)),
        ],
        out_specs=[blk3() for _ in range(n_out)],
        out_shape=([jax.ShapeDtypeStruct((bb, t, dim), jnp.float32)] * 2
                   + [jax.ShapeDtypeStruct((bb, t, dim), jnp.bfloat16)]
                   * (n_out - 2)),
        compiler_params=pltpu.CompilerParams(
            dimension_semantics=("parallel", "parallel")),
    )(x1, x2, o, wout_h, bout, g2, b2, w1, bias1, w2, bias2, gn, bn)


def _head(x1, x2, nf_g, nf_b, wf, bf, wc, bc):
    bb, t, dim = x1.shape
    hid = wf.shape[1]
    nc = wc.shape[1]

    def kern(x1_ref, x2_ref, g_ref, b_ref, wf_ref, bf_ref, wc_ref, bc_ref,
             o_ref):
        h = (x1_ref[...] + x2_ref[...]) * 0.5
        h = _layernorm_in(h, g_ref[...], b_ref[...])
        hm = jnp.mean(h, axis=1)
        f = jnp.maximum(jnp.dot(hm, wf_ref[...],
                                preferred_element_type=jnp.float32)
                        + bf_ref[...], 0.0)
        o_ref[...] = jnp.dot(f, wc_ref[...],
                             preferred_element_type=jnp.float32) + bc_ref[...]

    return pl.pallas_call(
        kern,
        in_specs=[
            pl.BlockSpec((bb, t, dim), lambda: (0, 0, 0)),
            pl.BlockSpec((bb, t, dim), lambda: (0, 0, 0)),
            pl.BlockSpec((1, dim), lambda: (0, 0)),
            pl.BlockSpec((1, dim), lambda: (0, 0)),
            pl.BlockSpec((dim, hid), lambda: (0, 0)),
            pl.BlockSpec((1, hid), lambda: (0, 0)),
            pl.BlockSpec((hid, nc), lambda: (0, 0)),
            pl.BlockSpec((1, nc), lambda: (0, 0)),
        ],
        out_specs=pl.BlockSpec((bb, nc), lambda: (0, 0)),
        out_shape=jax.ShapeDtypeStruct((bb, nc), jnp.float32),
    )(x1, x2, nf_g, nf_b, wf, bf, wc, bc)


def kernel(x, emb, pos, ln1_g, ln1_b, Wqk, Wv, Wout, bout, ln2_g, ln2_b,
           W1, b1, W2, b2, nf_g, nf_b, Wf, bf, Wc, bc):
    bb, t = x.shape
    dim = emb.shape[1]
    ll, _, hdh = Wqk.shape
    dh = 64
    nh = hdh // dh

    idx = x.reshape(bb * t).astype(jnp.int32)
    g = _sc_gather(emb, idx).reshape(bb, t, dim)
    h, hn = _embed(g, pos[:t], ln1_g[0].reshape(1, dim),
                   ln1_b[0].reshape(1, dim))

    wqk_h = (Wqk * (dh ** -0.5)).reshape(ll, dim, nh, dh).transpose(0, 2, 1, 3)
    wv_h = Wv.reshape(ll, dim, nh, dh).transpose(0, 2, 1, 3)
    wv_aug = jnp.concatenate([wv_h, jnp.zeros_like(wv_h)], axis=-1)
    wout_h = Wout.reshape(ll, nh, dh, dim)

    x1, x2 = h, h
    for l in range(ll):
        o = _attn_heads(hn, wqk_h[l], wv_aug[l])
        last = l == ll - 1
        gn = ln1_g[0 if last else l + 1].reshape(1, dim)
        bn = ln1_b[0 if last else l + 1].reshape(1, dim)
        outs = _merge(x1, x2, o, wout_h[l], bout[l].reshape(1, dim),
                      ln2_g[l].reshape(1, dim), ln2_b[l].reshape(1, dim),
                      W1[l], b1[l].reshape(1, -1), W2[l],
                      b2[l].reshape(1, dim), gn, bn, tblk=1024,
                      emit_ln=not last)
        x1, x2 = outs[0], outs[1]
        if not last:
            hn = outs[2]

    return _head(x1, x2, nf_g.reshape(1, dim), nf_b.reshape(1, dim),
                 Wf, bf.reshape(1, -1), Wc, bc.reshape(1, -1))


# speedup vs baseline: 8.1846x; 1.0261x over previous
import jax
import jax.numpy as jnp
from jax.experimental import pallas as pl
from jax.experimental.pallas import tpu as pltpu
from jax.experimental.pallas import tpu_sc as plsc


def _sc_gather(emb, idx_flat):
    n = idx_flat.shape[0]
    dim = emb.shape[1]
    window = 128
    indices = idx_flat.reshape(1, n)
    mesh = plsc.VectorSubcoreMesh(core_axis_name="c", subcore_axis_name="s")

    @pl.kernel(out_type=jax.ShapeDtypeStruct((n, dim), emb.dtype), mesh=mesh)
    def gather_kernel(emb_hbm, i_hbm, o_hbm):
        def body(i_vmem, o_vmem):
            pltpu.sync_copy(emb_hbm.at[i_vmem.at[0]], o_vmem)

        pltpu.emit_pipeline(
            body,
            grid=(n // window,),
            in_specs=[pl.BlockSpec((1, window), lambda i: (0, i))],
            out_specs=[pl.BlockSpec((window, dim), lambda i: (i, 0))],
            core_axis_name=("c", "s"),
            dimension_semantics=(pltpu.PARALLEL,),
        )(i_hbm, o_hbm)

    return gather_kernel(emb, indices)


def _embed(g, pos, g1, b1):
    bb, t, dim = g.shape

    def kern(g_ref, p_ref, g1_ref, b1_ref, h_ref, hn_ref):
        h = g_ref[0] + p_ref[...]
        h_ref[0] = h
        hn_ref[0] = _layernorm_in(h, g1_ref[...], b1_ref[...]).astype(
            jnp.bfloat16)

    return pl.pallas_call(
        kern,
        grid=(bb,),
        in_specs=[
            pl.BlockSpec((1, t, dim), lambda i: (i, 0, 0)),
            pl.BlockSpec((t, dim), lambda i: (0, 0)),
            pl.BlockSpec((1, dim), lambda i: (0, 0)),
            pl.BlockSpec((1, dim), lambda i: (0, 0)),
        ],
        out_specs=[
            pl.BlockSpec((1, t, dim), lambda i: (i, 0, 0)),
            pl.BlockSpec((1, t, dim), lambda i: (i, 0, 0)),
        ],
        out_shape=[
            jax.ShapeDtypeStruct((bb, t, dim), jnp.float32),
            jax.ShapeDtypeStruct((bb, t, dim), jnp.bfloat16),
        ],
        compiler_params=pltpu.CompilerParams(
            dimension_semantics=("parallel",)),
    )(g, pos, g1, b1)


def _layernorm_in(x, g, b):
    mu = jnp.mean(x, axis=-1, keepdims=True)
    var = jnp.mean((x - mu) ** 2, axis=-1, keepdims=True)
    return (x - mu) * jax.lax.rsqrt(var + 1e-5) * g + b


def _attn_heads(hn, wqk_h, wv_aug):
    bb, t, dim = hn.shape
    nh, _, dh2 = wv_aug.shape
    dh = dh2 // 2

    def kern(x_ref, wqk_ref, wv_ref, o_ref):
        for b in range(bb):
            hb = x_ref[b]
            qk = jnp.dot(hb, wqk_ref[0].astype(jnp.bfloat16),
                         preferred_element_type=jnp.float32)
            v_aug = jnp.dot(hb, wv_ref[0].astype(jnp.bfloat16),
                            preferred_element_type=jnp.float32)
            v_aug = v_aug + jnp.concatenate(
                [jnp.zeros((1, dh), jnp.float32),
                 jnp.ones((1, dh), jnp.float32)], axis=-1)
            norm = jnp.sqrt(jnp.sum(qk * qk, axis=-1, keepdims=True))
            kb = (qk / jnp.maximum(norm, 1e-13)).astype(jnp.bfloat16)
            s = jax.lax.dot_general(
                qk.astype(jnp.bfloat16), kb,
                (((1,), (1,)), ((), ())), preferred_element_type=jnp.float32)
            e = jnp.exp(s.astype(jnp.bfloat16))
            o_aug = jnp.dot(e, v_aug.astype(jnp.bfloat16),
                            preferred_element_type=jnp.float32)
            e_diag = jnp.exp(norm)
            num = o_aug[:, :dh] - e_diag * v_aug[:, :dh]
            den = o_aug[:, dh:dh + 1] - e_diag
            o_ref[b, 0] = (num / den).astype(jnp.bfloat16)

    return pl.pallas_call(
        kern,
        grid=(nh,),
        in_specs=[
            pl.BlockSpec((bb, t, dim), lambda j: (0, 0, 0)),
            pl.BlockSpec((1, dim, dh), lambda j: (j, 0, 0)),
            pl.BlockSpec((1, dim, 2 * dh), lambda j: (j, 0, 0)),
        ],
        out_specs=pl.BlockSpec((bb, 1, t, dh), lambda j: (0, j, 0, 0)),
        out_shape=jax.ShapeDtypeStruct((bb, nh, t, dh), jnp.bfloat16),
        compiler_params=pltpu.CompilerParams(
            dimension_semantics=("parallel",)),
    )(hn, wqk_h, wv_aug)


def _merge(x1, x2, o, wout_h, bout, g2, b2, w1, bias1, w2, bias2,
           gn, bn, tblk, emit_ln):
    bb, t, dim = x1.shape
    nh, dh, _ = wout_h.shape
    ff = w1.shape[1]

    def kern(x1_ref, x2_ref, o_ref, wout_ref, bout_ref, g2_ref, b2_ref,
             w1_ref, b1_ref, w2_ref, b2b_ref, gn_ref, bn_ref, *out_refs):
        acc = jnp.zeros((tblk, dim), jnp.float32)
        for h in range(nh):
            acc = acc + jnp.dot(o_ref[0, h],
                                wout_ref[h].astype(jnp.bfloat16),
                                preferred_element_type=jnp.float32)
        y1 = x1_ref[0] + acc + bout_ref[...]
        hh = _layernorm_in(y1, g2_ref[...], b2_ref[...])
        hid = jnp.dot(hh.astype(jnp.bfloat16), w1_ref[...].astype(jnp.bfloat16),
                      preferred_element_type=jnp.float32) + b1_ref[...]
        hid = 0.5 * hid * (1.0 + jax.lax.erf(hid * (2.0 ** -0.5)))
        y2 = x2_ref[0] + jnp.dot(hid.astype(jnp.bfloat16),
                                 w2_ref[...].astype(jnp.bfloat16),
                                 preferred_element_type=jnp.float32) + b2b_ref[...]
        out_refs[0][0] = y1
        out_refs[1][0] = y2
        if emit_ln:
            out_refs[2][0] = _layernorm_in(
                y2, gn_ref[...], bn_ref[...]).astype(jnp.bfloat16)

    nblk = t // tblk
    n_out = 3 if emit_ln else 2
    blk3 = lambda: pl.BlockSpec((1, tblk, dim), lambda i, j: (i, j, 0))
    return pl.pallas_call(
        kern,
        grid=(bb, nblk),
        in_specs=[
            blk3(),
            blk3(),
            pl.BlockSpec((1, nh, tblk, dh), lambda i, j: (i, 0, j, 0)),
            pl.BlockSpec((nh, dh, dim), lambda i, j: (0, 0, 0)),
            pl.BlockSpec((1, dim), lambda i, j: (0, 0)),
            pl.BlockSpec((1, dim), lambda i, j: (0, 0)),
            pl.BlockSpec((1, dim), lambda i, j: (0, 0)),
            pl.BlockSpec((dim, ff), lambda i, j: (0, 0)),
            pl.BlockSpec((1, ff), lambda i, j: (0, 0)),
            pl.BlockSpec((ff, dim), lambda i, j: (0, 0)),
            pl.BlockSpec((1, dim), lambda i, j: (0, 0)),
            pl.BlockSpec((1, dim), lambda i, j: (0, 0)),
            pl.BlockSpec((1, dim), lambda i, j: (0, 0)),
        ],
        out_specs=[blk3() for _ in range(n_out)],
        out_shape=([jax.ShapeDtypeStruct((bb, t, dim), jnp.float32)] * 2
                   + [jax.ShapeDtypeStruct((bb, t, dim), jnp.bfloat16)]
                   * (n_out - 2)),
        compiler_params=pltpu.CompilerParams(
            dimension_semantics=("parallel", "parallel")),
    )(x1, x2, o, wout_h, bout, g2, b2, w1, bias1, w2, bias2, gn, bn)


def _merge_head(x1, x2, o, wout_h, bout, g2, b2, w1, bias1, w2, bias2,
                nf_g, nf_b, wf, bfv, wc, bcv):
    bb, t, dim = x1.shape
    nh, dh, _ = wout_h.shape
    ff = w1.shape[1]
    nc = wc.shape[1]
    hid_d = wf.shape[1]

    def kern(x1_ref, x2_ref, o_ref, wout_ref, bout_ref, g2_ref, b2_ref,
             w1_ref, b1_ref, w2_ref, b2b_ref, nfg_ref, nfb_ref, wf_ref,
             bf_ref, wc_ref, bc_ref, out_ref):
        acc = jnp.zeros((t, dim), jnp.float32)
        for h in range(nh):
            acc = acc + jnp.dot(o_ref[0, h],
                                wout_ref[h].astype(jnp.bfloat16),
                                preferred_element_type=jnp.float32)
        y1 = x1_ref[0] + acc + bout_ref[...]
        hh = _layernorm_in(y1, g2_ref[...], b2_ref[...])
        hid = jnp.dot(hh.astype(jnp.bfloat16), w1_ref[...].astype(jnp.bfloat16),
                      preferred_element_type=jnp.float32) + b1_ref[...]
        hid = 0.5 * hid * (1.0 + jax.lax.erf(hid * (2.0 ** -0.5)))
        y2 = x2_ref[0] + jnp.dot(hid.astype(jnp.bfloat16),
                                 w2_ref[...].astype(jnp.bfloat16),
                                 preferred_element_type=jnp.float32) + b2b_ref[...]
        hfin = _layernorm_in((y1 + y2) * 0.5, nfg_ref[...], nfb_ref[...])
        hm = jnp.mean(hfin, axis=0, keepdims=True)
        f = jnp.maximum(jnp.dot(hm, wf_ref[...],
                                preferred_element_type=jnp.float32)
                        + bf_ref[...], 0.0)
        out_ref[0] = jnp.dot(f, wc_ref[...],
                             preferred_element_type=jnp.float32) + bc_ref[...]

    vec = lambda: pl.BlockSpec((1, dim), lambda i: (0, 0))
    return pl.pallas_call(
        kern,
        grid=(bb,),
        in_specs=[
            pl.BlockSpec((1, t, dim), lambda i: (i, 0, 0)),
            pl.BlockSpec((1, t, dim), lambda i: (i, 0, 0)),
            pl.BlockSpec((1, nh, t, dh), lambda i: (i, 0, 0, 0)),
            pl.BlockSpec((nh, dh, dim), lambda i: (0, 0, 0)),
            vec(), vec(), vec(),
            pl.BlockSpec((dim, ff), lambda i: (0, 0)),
            pl.BlockSpec((1, ff), lambda i: (0, 0)),
            pl.BlockSpec((ff, dim), lambda i: (0, 0)),
            vec(), vec(), vec(),
            pl.BlockSpec((dim, hid_d), lambda i: (0, 0)),
            pl.BlockSpec((1, hid_d), lambda i: (0, 0)),
            pl.BlockSpec((hid_d, nc), lambda i: (0, 0)),
            pl.BlockSpec((1, nc), lambda i: (0, 0)),
        ],
        out_specs=pl.BlockSpec((1, 1, nc), lambda i: (i, 0, 0)),
        out_shape=jax.ShapeDtypeStruct((bb, 1, nc), jnp.float32),
        compiler_params=pltpu.CompilerParams(
            dimension_semantics=("arbitrary",)),
    )(x1, x2, o, wout_h, bout, g2, b2, w1, bias1, w2, bias2,
      nf_g, nf_b, wf, bfv, wc, bcv)


def _head(x1, x2, nf_g, nf_b, wf, bf, wc, bc):
    bb, t, dim = x1.shape
    hid = wf.shape[1]
    nc = wc.shape[1]

    def kern(x1_ref, x2_ref, g_ref, b_ref, wf_ref, bf_ref, wc_ref, bc_ref,
             o_ref):
        h = (x1_ref[...] + x2_ref[...]) * 0.5
        h = _layernorm_in(h, g_ref[...], b_ref[...])
        hm = jnp.mean(h, axis=1)
        f = jnp.maximum(jnp.dot(hm, wf_ref[...],
                                preferred_element_type=jnp.float32)
                        + bf_ref[...], 0.0)
        o_ref[...] = jnp.dot(f, wc_ref[...],
                             preferred_element_type=jnp.float32) + bc_ref[...]

    return pl.pallas_call(
        kern,
        in_specs=[
            pl.BlockSpec((bb, t, dim), lambda: (0, 0, 0)),
            pl.BlockSpec((bb, t, dim), lambda: (0, 0, 0)),
            pl.BlockSpec((1, dim), lambda: (0, 0)),
            pl.BlockSpec((1, dim), lambda: (0, 0)),
            pl.BlockSpec((dim, hid), lambda: (0, 0)),
            pl.BlockSpec((1, hid), lambda: (0, 0)),
            pl.BlockSpec((hid, nc), lambda: (0, 0)),
            pl.BlockSpec((1, nc), lambda: (0, 0)),
        ],
        out_specs=pl.BlockSpec((bb, nc), lambda: (0, 0)),
        out_shape=jax.ShapeDtypeStruct((bb, nc), jnp.float32),
    )(x1, x2, nf_g, nf_b, wf, bf, wc, bc)


def kernel(x, emb, pos, ln1_g, ln1_b, Wqk, Wv, Wout, bout, ln2_g, ln2_b,
           W1, b1, W2, b2, nf_g, nf_b, Wf, bf, Wc, bc):
    bb, t = x.shape
    dim = emb.shape[1]
    ll, _, hdh = Wqk.shape
    dh = 64
    nh = hdh // dh

    idx = x.reshape(bb * t).astype(jnp.int32)
    g = _sc_gather(emb, idx).reshape(bb, t, dim)
    h, hn = _embed(g, pos[:t], ln1_g[0].reshape(1, dim),
                   ln1_b[0].reshape(1, dim))

    wqk_h = (Wqk * (dh ** -0.5)).reshape(ll, dim, nh, dh).transpose(0, 2, 1, 3)
    wv_h = Wv.reshape(ll, dim, nh, dh).transpose(0, 2, 1, 3)
    wv_aug = jnp.concatenate([wv_h, jnp.zeros_like(wv_h)], axis=-1)
    wout_h = Wout.reshape(ll, nh, dh, dim)

    x1, x2 = h, h
    for l in range(ll - 1):
        o = _attn_heads(hn, wqk_h[l], wv_aug[l])
        outs = _merge(x1, x2, o, wout_h[l], bout[l].reshape(1, dim),
                      ln2_g[l].reshape(1, dim), ln2_b[l].reshape(1, dim),
                      W1[l], b1[l].reshape(1, -1), W2[l],
                      b2[l].reshape(1, dim), ln1_g[l + 1].reshape(1, dim),
                      ln1_b[l + 1].reshape(1, dim), tblk=1024, emit_ln=True)
        x1, x2, hn = outs[0], outs[1], outs[2]

    lz = ll - 1
    o = _attn_heads(hn, wqk_h[lz], wv_aug[lz])
    return _merge_head(
        x1, x2, o, wout_h[lz], bout[lz].reshape(1, dim),
        ln2_g[lz].reshape(1, dim), ln2_b[lz].reshape(1, dim), W1[lz],
        b1[lz].reshape(1, -1), W2[lz], b2[lz].reshape(1, dim),
        nf_g.reshape(1, dim), nf_b.reshape(1, dim), Wf, bf.reshape(1, -1),
        Wc, bc.reshape(1, -1)).reshape(bb, -1)
